# baseline (speedup 1.0000x reference)
import functools
import jax
import jax.numpy as jnp
from jax import lax
from jax.experimental import pallas as pl
from jax.experimental.pallas import tpu as pltpu
from jax.experimental.pallas import tpu_sc as plsc


def kernel(x, pos_embedding):
    S, N = x.shape
    D = pos_embedding.shape[1]
    info = plsc.get_sparse_core_info()
    NC, NS = info.num_cores, info.num_subcores
    NW = NC * NS
    rows_per_w = S // NW
    mesh = plsc.VectorSubcoreMesh(core_axis_name="c", subcore_axis_name="s")

    C = 32
    chunks = rows_per_w // C
    NBUF = 4

    @functools.partial(
        pl.kernel,
        out_type=jax.ShapeDtypeStruct((S, N, D), pos_embedding.dtype),
        mesh=mesh,
        scratch_types=(
            [pltpu.VMEM((C, D), pos_embedding.dtype) for _ in range(NBUF)]
            + [pltpu.SemaphoreType.DMA for _ in range(2 * NBUF)]
        ),
    )
    def broadcast_rows(table_hbm, out_hbm, *scratch):
        bufs = scratch[:NBUF]
        gsems = scratch[NBUF : 2 * NBUF]
        ssems = scratch[2 * NBUF :]
        wid = lax.axis_index("s") * NC + lax.axis_index("c")
        base = wid * rows_per_w

        gh = [None] * NBUF
        sh = [None] * NBUF
        for k in range(min(NBUF, chunks)):
            gh[k] = pltpu.async_copy(
                table_hbm.at[pl.ds(base + k * C, C)], bufs[k], gsems[k]
            )
        for k in range(chunks):
            b = k % NBUF
            gh[b].wait()
            s0 = base + k * C
            sh[b] = [
                pltpu.async_copy(bufs[b], out_hbm.at[pl.ds(s0, C), n], ssems[b])
                for n in range(N)
            ]
            nk = k + NBUF
            if nk < chunks:
                for h in sh[b]:
                    h.wait()
                gh[b] = pltpu.async_copy(
                    table_hbm.at[pl.ds(base + nk * C, C)], bufs[b], gsems[b]
                )
        for b in range(NBUF):
            if sh[b] is not None:
                for h in sh[b]:
                    h.wait()

    return broadcast_rows(pos_embedding)


# speedup vs baseline: 1.0307x; 1.0307x over previous
import functools
import jax
import jax.numpy as jnp
from jax import lax
from jax.experimental import pallas as pl
from jax.experimental.pallas import tpu as pltpu
from jax.experimental.pallas import tpu_sc as plsc


def kernel(x, pos_embedding):
    S, N = x.shape
    D = pos_embedding.shape[1]
    info = plsc.get_sparse_core_info()
    NC, NS = info.num_cores, info.num_subcores
    NW = NC * NS
    rows_per_w = S // NW
    mesh = plsc.VectorSubcoreMesh(core_axis_name="c", subcore_axis_name="s")

    C = 64
    chunks = rows_per_w // C
    NBUF = 2

    @functools.partial(
        pl.kernel,
        out_type=jax.ShapeDtypeStruct((S, N, D), pos_embedding.dtype),
        mesh=mesh,
        scratch_types=(
            [pltpu.VMEM((C, D), pos_embedding.dtype) for _ in range(NBUF)]
            + [pltpu.SemaphoreType.DMA for _ in range(2 * NBUF)]
        ),
    )
    def broadcast_rows(table_hbm, out_hbm, *scratch):
        bufs = scratch[:NBUF]
        gsems = scratch[NBUF : 2 * NBUF]
        ssems = scratch[2 * NBUF :]
        wid = lax.axis_index("s") * NC + lax.axis_index("c")
        base = wid * rows_per_w

        gh = [None] * NBUF
        sh = [None] * NBUF
        for k in range(min(NBUF, chunks)):
            gh[k] = pltpu.async_copy(
                table_hbm.at[pl.ds(base + k * C, C)], bufs[k], gsems[k]
            )
        for k in range(chunks):
            b = k % NBUF
            gh[b].wait()
            s0 = base + k * C
            sh[b] = [
                pltpu.async_copy(bufs[b], out_hbm.at[pl.ds(s0, C), n], ssems[b])
                for n in range(N)
            ]
            nk = k + NBUF
            if nk < chunks:
                for h in sh[b]:
                    h.wait()
                gh[b] = pltpu.async_copy(
                    table_hbm.at[pl.ds(base + nk * C, C)], bufs[b], gsems[b]
                )
        for b in range(NBUF):
            if sh[b] is not None:
                for h in sh[b]:
                    h.wait()

    return broadcast_rows(pos_embedding)


# speedup vs baseline: 1.0388x; 1.0079x over previous
import functools
import jax
import jax.numpy as jnp
from jax import lax
from jax.experimental import pallas as pl
from jax.experimental.pallas import tpu as pltpu
from jax.experimental.pallas import tpu_sc as plsc


def kernel(x, pos_embedding):
    S, N = x.shape
    D = pos_embedding.shape[1]
    info = plsc.get_sparse_core_info()
    NC, NS = info.num_cores, info.num_subcores
    NW = NC * NS
    rows_per_w = S // NW
    mesh = plsc.VectorSubcoreMesh(core_axis_name="c", subcore_axis_name="s")

    C = 128
    chunks = rows_per_w // C
    NBUF = 1

    @functools.partial(
        pl.kernel,
        out_type=jax.ShapeDtypeStruct((S, N, D), pos_embedding.dtype),
        mesh=mesh,
        scratch_types=(
            [pltpu.VMEM((C, D), pos_embedding.dtype) for _ in range(NBUF)]
            + [pltpu.SemaphoreType.DMA for _ in range(2 * NBUF)]
        ),
    )
    def broadcast_rows(table_hbm, out_hbm, *scratch):
        bufs = scratch[:NBUF]
        gsems = scratch[NBUF : 2 * NBUF]
        ssems = scratch[2 * NBUF :]
        wid = lax.axis_index("s") * NC + lax.axis_index("c")
        base = wid * rows_per_w

        gh = [None] * NBUF
        sh = [None] * NBUF
        for k in range(min(NBUF, chunks)):
            gh[k] = pltpu.async_copy(
                table_hbm.at[pl.ds(base + k * C, C)], bufs[k], gsems[k]
            )
        for k in range(chunks):
            b = k % NBUF
            gh[b].wait()
            s0 = base + k * C
            sh[b] = [
                pltpu.async_copy(bufs[b], out_hbm.at[pl.ds(s0, C), n], ssems[b])
                for n in range(N)
            ]
            nk = k + NBUF
            if nk < chunks:
                for h in sh[b]:
                    h.wait()
                gh[b] = pltpu.async_copy(
                    table_hbm.at[pl.ds(base + nk * C, C)], bufs[b], gsems[b]
                )
        for b in range(NBUF):
            if sh[b] is not None:
                for h in sh[b]:
                    h.wait()

    return broadcast_rows(pos_embedding)


# speedup vs baseline: 1.0417x; 1.0027x over previous
import functools
import jax
from jax import lax
from jax.experimental import pallas as pl
from jax.experimental.pallas import tpu as pltpu
from jax.experimental.pallas import tpu_sc as plsc


def kernel(x, pos_embedding):
    S, N = x.shape
    D = pos_embedding.shape[1]
    info = plsc.get_sparse_core_info()
    NC = info.num_cores
    NW = NC * info.num_subcores
    R = S // NW

    mesh = plsc.VectorSubcoreMesh(core_axis_name="c", subcore_axis_name="s")

    @functools.partial(
        pl.kernel,
        out_type=jax.ShapeDtypeStruct((S, N, D), pos_embedding.dtype),
        mesh=mesh,
        scratch_types=[
            pltpu.VMEM((R, D), pos_embedding.dtype),
            pltpu.SemaphoreType.DMA,
        ],
    )
    def broadcast_rows(table_hbm, out_hbm, buf, sem):
        wid = lax.axis_index("s") * NC + lax.axis_index("c")
        base = wid * R
        pltpu.sync_copy(table_hbm.at[pl.ds(base, R)], buf)
        copies = [
            pltpu.async_copy(buf, out_hbm.at[pl.ds(base, R), n], sem)
            for n in range(N)
        ]
        for cp in copies:
            cp.wait()

    return broadcast_rows(pos_embedding)
